# baseline (speedup 1.0000x reference)
import jax
import jax.numpy as jnp
from jax import lax
from jax.experimental import pallas as pl
from jax.experimental.pallas import tpu as pltpu
from jax.experimental.pallas import tpu_sc as plsc

D_MODEL = 64
SCALE = 8.0

NUM_CORES = 2
NUM_SUBCORES = 16
LANES = 16
NUM_WORKERS = NUM_CORES * NUM_SUBCORES

CHUNK = 128


def _emb_body(x_hbm, table_hbm, out_hbm, idx_v, rows_v, gsem):
    wid = lax.axis_index("s") * NUM_CORES + lax.axis_index("c")
    bpw = x_hbm.shape[0] // NUM_WORKERS
    base = wid * bpw
    pltpu.sync_copy(x_hbm.at[pl.ds(base, bpw)], idx_v)

    def chunk_body(ci, carry):
        cbase = ci * CHUNK
        pltpu.async_copy(
            table_hbm.at[idx_v.at[pl.ds(cbase, CHUNK)]], rows_v, gsem
        ).wait()

        def group_body(g, carry2):
            idxvec = idx_v[pl.ds(cbase + g * LANES, LANES)]
            svec = jnp.where(idxvec == 0, 0.0, SCALE).astype(jnp.float32)
            dnums = lax.GatherDimensionNumbers(
                offset_dims=(), collapsed_slice_dims=(0,),
                start_index_map=(0,))
            for r in range(LANES):
                splat = lax.gather(
                    svec, jnp.full((LANES, 1), r, jnp.int32), dnums,
                    slice_sizes=(1,),
                    mode=lax.GatherScatterMode.PROMISE_IN_BOUNDS)
                row = g * LANES + r
                for cb in range(D_MODEL // LANES):
                    sl = pl.ds(cb * LANES, LANES)
                    rows_v[row, sl] = rows_v[row, sl] * splat
            return carry2

        lax.fori_loop(0, CHUNK // LANES, group_body, 0)
        pltpu.sync_copy(rows_v, out_hbm.at[pl.ds(base + cbase, CHUNK)])
        return carry

    lax.fori_loop(0, bpw // CHUNK, chunk_body, 0)


def kernel(x, table):
    rows, cols = x.shape
    b = rows * cols
    xf = x.reshape(b)
    k = pl.kernel(
        _emb_body,
        out_type=jax.ShapeDtypeStruct((b, D_MODEL), jnp.float32),
        mesh=plsc.VectorSubcoreMesh(
            core_axis_name="c", subcore_axis_name="s"),
        scratch_types=[
            pltpu.VMEM((b // NUM_WORKERS,), jnp.int32),
            pltpu.VMEM((CHUNK, D_MODEL), jnp.float32),
            pltpu.SemaphoreType.DMA,
        ],
        compiler_params=pltpu.CompilerParams(use_tc_tiling_on_sc=False),
    )
    out = k(xf, table)
    return out.reshape(rows, cols, D_MODEL)
